# Initial kernel scaffold; baseline (speedup 1.0000x reference)
import jax
import jax.numpy as jnp
from jax.experimental import pallas as pl


def kernel(userid, itemid, user_age, gender, user_occupation, item_kind, label, user_table, item_table, age_table, gender_table, occ_table, kind_table, W1, b1, W2, b2, W3, b3, W4, b4):
    raise NotImplementedError("write your pallas kernel here")


# speedup vs baseline: 2.0430x; 2.0430x over previous
import functools

import jax
import jax.numpy as jnp
from jax import lax
from jax.experimental import pallas as pl
from jax.experimental.pallas import tpu as pltpu
from jax.experimental.pallas import tpu_sc as plsc

_B = 4096
_ED = 65
_DIM = 64
_BB = 512
_NBLK = _B // _BB


def _sc_dims():
    try:
        info = plsc.get_sparse_core_info()
        return info.num_cores, info.num_subcores
    except Exception:
        return 2, 16


def _make_sc_gather():
    nc, ns = _sc_dims()
    nw = nc * ns
    bpw = _B // nw
    epw = bpw * _ED
    nchunk = epw // 16
    ntr = epw // 128

    mesh = plsc.VectorSubcoreMesh(core_axis_name="c", subcore_axis_name="s")

    @functools.partial(
        pl.kernel,
        mesh=mesh,
        out_type=[
            jax.ShapeDtypeStruct((_B * _ED,), jnp.float32),
            jax.ShapeDtypeStruct((_B * _ED,), jnp.float32),
        ],
        scratch_types=[
            pltpu.VMEM((bpw,), jnp.int32),
            pltpu.VMEM((bpw,), jnp.int32),
            pltpu.VMEM((epw,), jnp.int32),
            pltpu.VMEM((epw,), jnp.int32),
            pltpu.VMEM((epw,), jnp.float32),
            pltpu.VMEM((epw,), jnp.float32),
            pltpu.SemaphoreType.DMA,
            pltpu.SemaphoreType.DMA,
        ],
        compiler_params=pltpu.CompilerParams(needs_layout_passes=False),
    )
    def sc_gather(uidx_hbm, iidx_hbm, utab_hbm, itab_hbm, uout_hbm, iout_hbm,
                  uidx_v, iidx_v, ueidx_v, ieidx_v, urows_v, irows_v, usem, isem):
        wid = lax.axis_index("s") * nc + lax.axis_index("c")
        base = wid * bpw
        pltpu.sync_copy(uidx_hbm.at[pl.ds(base, bpw)], uidx_v)
        pltpu.sync_copy(iidx_hbm.at[pl.ds(base, bpw)], iidx_v)

        iota = lax.iota(jnp.int32, 16)

        def build(i, idx_v, eidx_v):
            q = lax.add(lax.broadcast(lax.mul(i, 16), (16,)), iota)
            row = lax.div(q, lax.broadcast(_ED, (16,)))
            c = lax.sub(q, lax.mul(row, lax.broadcast(_ED, (16,))))
            rv = plsc.load_gather(idx_v, [row])
            eidx_v[pl.ds(i * 16, 16)] = lax.add(lax.mul(rv, lax.broadcast(_ED, (16,))), c)

        def ubody(i):
            build(i, uidx_v, ueidx_v)

        def ibody(i):
            build(i, iidx_v, ieidx_v)

        lax.fori_loop(0, nchunk, lambda i, _: (ubody(i), 0)[1], 0)
        ucs = [
            pltpu.async_copy(
                utab_hbm.at[ueidx_v.at[pl.ds(t * 128, 128)]],
                urows_v.at[pl.ds(t * 128, 128)], usem)
            for t in range(ntr)
        ]
        lax.fori_loop(0, nchunk, lambda i, _: (ibody(i), 0)[1], 0)
        ics = [
            pltpu.async_copy(
                itab_hbm.at[ieidx_v.at[pl.ds(t * 128, 128)]],
                irows_v.at[pl.ds(t * 128, 128)], isem)
            for t in range(ntr)
        ]
        for c in ucs:
            c.wait()
        pltpu.sync_copy(urows_v, uout_hbm.at[pl.ds(wid * epw, epw)])
        for c in ics:
            c.wait()
        pltpu.sync_copy(irows_v, iout_hbm.at[pl.ds(wid * epw, epw)])

    return sc_gather


def _tc_body(uw_ref, iw_ref, age_ref, gen_ref, occ_ref, kind_ref, lab_ref,
             at_ref, gt_ref, ot_ref, kt_ref,
             w1_ref, b1_ref, w2_ref, b2_ref, w3_ref, b3_ref, w4_ref, b4_ref,
             loss_ref, p_ref):
    i = pl.program_id(0)
    f32 = jnp.float32

    uw = uw_ref[...]
    iw = iw_ref[...]
    age = age_ref[...]
    gen = gen_ref[...]
    occ = occ_ref[...]
    kid = kind_ref[...]

    at = at_ref[...]
    gt = gt_ref[...]
    ot = ot_ref[...]
    kt = kt_ref[...]

    aoh = (age == lax.broadcasted_iota(jnp.int32, (_BB, 8), 1)).astype(f32)
    goh = (gen == lax.broadcasted_iota(jnp.int32, (_BB, 3), 1)).astype(f32)
    ooh = (occ == lax.broadcasted_iota(jnp.int32, (_BB, 25), 1)).astype(f32)
    aw = jnp.dot(aoh, at, preferred_element_type=f32)
    gw = jnp.dot(goh, gt, preferred_element_type=f32)
    ow = jnp.dot(ooh, ot, preferred_element_type=f32)

    kiota = lax.broadcasted_iota(jnp.int32, (_BB, 20), 1)
    kws = []
    for j in range(10):
        kj = kid[:, j:j + 1]
        koh = ((kj == kiota) & (kj != 0)).astype(f32)
        kws.append(jnp.dot(koh, kt, preferred_element_type=f32))

    fields = [uw, iw, aw, ow] + kws + [gw]
    one = fields[0][:, 0:1]
    for fld in fields[1:]:
        one = one + fld[:, 0:1]

    es = [fld[:, 1:] for fld in fields]
    s = es[0]
    for e in es[1:]:
        s = s + e
    sumsq = jnp.sum(es[0] * es[0], axis=1, keepdims=True)
    for e in es[1:]:
        sumsq = sumsq + jnp.sum(e * e, axis=1, keepdims=True)
    two = 0.5 * (jnp.sum(s * s, axis=1, keepdims=True) - sumsq)

    h0 = jnp.concatenate(es, axis=1)
    h = jnp.maximum(jnp.dot(h0, w1_ref[...], preferred_element_type=f32) + b1_ref[...], 0.0)
    h = jnp.maximum(jnp.dot(h, w2_ref[...], preferred_element_type=f32) + b2_ref[...], 0.0)
    h = jnp.maximum(jnp.dot(h, w3_ref[...], preferred_element_type=f32) + b3_ref[...], 0.0)
    m = jnp.dot(h, w4_ref[...], preferred_element_type=f32) + b4_ref[...]

    logit = one + two + m
    p = 1.0 / (1.0 + jnp.exp(-logit))
    p_ref[...] = p

    lab = lab_ref[...]
    ploss = jnp.sum(-(lab * jnp.log(p + 1e-6) + (1.0 - lab) * jnp.log(1.0 - p + 1e-6)),
                    axis=(0, 1), keepdims=True)

    @pl.when(i == 0)
    def _init():
        loss_ref[...] = jnp.zeros((1, 1), jnp.float32)

    loss_ref[...] += ploss

    @pl.when(i == _NBLK - 1)
    def _final():
        loss_ref[...] = loss_ref[...] * (1.0 / _B)


def _tc_main(uw, iw, age, gen, occ, kind, label,
             age_table, gender_table, occ_table, kind_table,
             W1, b1, W2, b2, W3, b3, W4, b4):
    bspec = lambda shp: pl.BlockSpec(shp, lambda i: (i, 0))
    fspec = lambda shp: pl.BlockSpec(shp, lambda i: (0, 0))
    grid_spec = pl.GridSpec(
        grid=(_NBLK,),
        in_specs=[
            bspec((_BB, _ED)), bspec((_BB, _ED)),
            bspec((_BB, 1)), bspec((_BB, 1)), bspec((_BB, 1)), bspec((_BB, 10)),
            bspec((_BB, 1)),
            fspec((8, _ED)), fspec((3, _ED)), fspec((25, _ED)), fspec((20, _ED)),
            fspec((960, 128)), fspec((1, 128)),
            fspec((128, 64)), fspec((1, 64)),
            fspec((64, 32)), fspec((1, 32)),
            fspec((32, 1)), fspec((1, 1)),
        ],
        out_specs=[
            fspec((1, 1)),
            bspec((_BB, 1)),
        ],
    )
    loss, p = pl.pallas_call(
        _tc_body,
        grid_spec=grid_spec,
        out_shape=[
            jax.ShapeDtypeStruct((1, 1), jnp.float32),
            jax.ShapeDtypeStruct((_B, 1), jnp.float32),
        ],
    )(uw, iw, age, gen, occ, kind, label,
      age_table, gender_table, occ_table, kind_table,
      W1, b1, W2, b2, W3, b3, W4, b4)
    return loss, p


def kernel(userid, itemid, user_age, gender, user_occupation, item_kind, label,
           user_table, item_table, age_table, gender_table, occ_table, kind_table,
           W1, b1, W2, b2, W3, b3, W4, b4):
    uidx = userid.reshape(_B).astype(jnp.int32)
    iidx = itemid.reshape(_B).astype(jnp.int32)
    sc_gather = _make_sc_gather()
    uw_flat, iw_flat = sc_gather(uidx, iidx,
                                 user_table.reshape(-1), item_table.reshape(-1))
    uw = uw_flat.reshape(_B, _ED)
    iw = iw_flat.reshape(_B, _ED)

    loss, p = _tc_main(
        uw, iw,
        user_age.astype(jnp.int32), gender.astype(jnp.int32),
        user_occupation.astype(jnp.int32), item_kind.astype(jnp.int32),
        label,
        age_table, gender_table, occ_table, kind_table,
        W1, b1.reshape(1, 128), W2, b2.reshape(1, 64),
        W3, b3.reshape(1, 32), W4, b4.reshape(1, 1),
    )
    return loss.reshape(()), p


# speedup vs baseline: 4.4684x; 2.1872x over previous
import functools

import jax
import jax.numpy as jnp
from jax import lax
from jax.experimental import pallas as pl
from jax.experimental.pallas import tpu as pltpu
from jax.experimental.pallas import tpu_sc as plsc

_B = 4096
_ED = 65
_DIM = 64
_BB = 512
_NBLK = _B // _BB


def _sc_dims():
    try:
        info = plsc.get_sparse_core_info()
        return info.num_cores, info.num_subcores
    except Exception:
        return 2, 16


def _make_sc_gather():
    nc, ns = _sc_dims()
    nw = nc * ns
    bpw = _B // nw
    epw = bpw * _ED
    nchunk = epw // 16
    ntr = epw // 128

    mesh = plsc.VectorSubcoreMesh(core_axis_name="c", subcore_axis_name="s")

    @functools.partial(
        pl.kernel,
        mesh=mesh,
        out_type=[
            jax.ShapeDtypeStruct((_B * _ED,), jnp.float32),
            jax.ShapeDtypeStruct((_B * _ED,), jnp.float32),
        ],
        scratch_types=[
            pltpu.VMEM((bpw,), jnp.int32),
            pltpu.VMEM((bpw,), jnp.int32),
            pltpu.VMEM((epw,), jnp.int32),
            pltpu.VMEM((epw,), jnp.int32),
            pltpu.VMEM((epw,), jnp.float32),
            pltpu.VMEM((epw,), jnp.float32),
            pltpu.SemaphoreType.DMA,
            pltpu.SemaphoreType.DMA,
        ],
        compiler_params=pltpu.CompilerParams(needs_layout_passes=False),
    )
    def sc_gather(uidx_hbm, iidx_hbm, utab_hbm, itab_hbm, uout_hbm, iout_hbm,
                  uidx_v, iidx_v, ueidx_v, ieidx_v, urows_v, irows_v, usem, isem):
        wid = lax.axis_index("s") * nc + lax.axis_index("c")
        base = wid * bpw
        pltpu.sync_copy(uidx_hbm.at[pl.ds(base, bpw)], uidx_v)
        pltpu.sync_copy(iidx_hbm.at[pl.ds(base, bpw)], iidx_v)

        iota = lax.iota(jnp.int32, 16)

        def build(i, idx_v, eidx_v, nrows):
            q = lax.add(lax.broadcast(lax.mul(i, 16), (16,)), iota)
            c = lax.div(q, lax.broadcast(bpw, (16,)))
            j = lax.sub(q, lax.mul(c, lax.broadcast(bpw, (16,))))
            rv = plsc.load_gather(idx_v, [j])
            eidx_v[pl.ds(i * 16, 16)] = lax.add(lax.mul(c, lax.broadcast(nrows, (16,))), rv)

        def ubody(i):
            build(i, uidx_v, ueidx_v, 100000)

        def ibody(i):
            build(i, iidx_v, ieidx_v, 100000)

        lax.fori_loop(0, nchunk, lambda i, _: (ubody(i), 0)[1], 0)
        ucs = [
            pltpu.async_copy(
                utab_hbm.at[ueidx_v.at[pl.ds(t * 128, 128)]],
                urows_v.at[pl.ds(t * 128, 128)], usem)
            for t in range(ntr)
        ]
        lax.fori_loop(0, nchunk, lambda i, _: (ibody(i), 0)[1], 0)
        ics = [
            pltpu.async_copy(
                itab_hbm.at[ieidx_v.at[pl.ds(t * 128, 128)]],
                irows_v.at[pl.ds(t * 128, 128)], isem)
            for t in range(ntr)
        ]
        for c in ucs:
            c.wait()
        pltpu.sync_copy(urows_v, uout_hbm.at[pl.ds(wid * epw, epw)])
        for c in ics:
            c.wait()
        pltpu.sync_copy(irows_v, iout_hbm.at[pl.ds(wid * epw, epw)])

    return sc_gather


def _tc_body(uw_ref, iw_ref, age_ref, gen_ref, occ_ref, kind_ref, lab_ref,
             at_ref, gt_ref, ot_ref, kt_ref,
             w1_ref, b1_ref, w2_ref, b2_ref, w3_ref, b3_ref, w4_ref, b4_ref,
             loss_ref, p_ref):
    i = pl.program_id(0)
    f32 = jnp.float32

    uw = uw_ref[...]
    iw = iw_ref[...]
    age = age_ref[...]
    gen = gen_ref[...]
    occ = occ_ref[...]
    kid = kind_ref[...]

    at = at_ref[...]
    gt = gt_ref[...]
    ot = ot_ref[...]
    kt = kt_ref[...]

    aoh = (age == lax.broadcasted_iota(jnp.int32, (_BB, 8), 1)).astype(f32)
    goh = (gen == lax.broadcasted_iota(jnp.int32, (_BB, 3), 1)).astype(f32)
    ooh = (occ == lax.broadcasted_iota(jnp.int32, (_BB, 25), 1)).astype(f32)
    aw = jnp.dot(aoh, at, preferred_element_type=f32)
    gw = jnp.dot(goh, gt, preferred_element_type=f32)
    ow = jnp.dot(ooh, ot, preferred_element_type=f32)

    kiota = lax.broadcasted_iota(jnp.int32, (_BB, 20), 1)
    kws = []
    for j in range(10):
        kj = kid[:, j:j + 1]
        koh = ((kj == kiota) & (kj != 0)).astype(f32)
        kws.append(jnp.dot(koh, kt, preferred_element_type=f32))

    fields = [uw, iw, aw, ow] + kws + [gw]
    one = fields[0][:, 0:1]
    for fld in fields[1:]:
        one = one + fld[:, 0:1]

    es = [fld[:, 1:] for fld in fields]
    s = es[0]
    for e in es[1:]:
        s = s + e
    sumsq = jnp.sum(es[0] * es[0], axis=1, keepdims=True)
    for e in es[1:]:
        sumsq = sumsq + jnp.sum(e * e, axis=1, keepdims=True)
    two = 0.5 * (jnp.sum(s * s, axis=1, keepdims=True) - sumsq)

    h0 = jnp.concatenate(es, axis=1)
    h = jnp.maximum(jnp.dot(h0, w1_ref[...], preferred_element_type=f32) + b1_ref[...], 0.0)
    h = jnp.maximum(jnp.dot(h, w2_ref[...], preferred_element_type=f32) + b2_ref[...], 0.0)
    h = jnp.maximum(jnp.dot(h, w3_ref[...], preferred_element_type=f32) + b3_ref[...], 0.0)
    m = jnp.dot(h, w4_ref[...], preferred_element_type=f32) + b4_ref[...]

    logit = one + two + m
    p = 1.0 / (1.0 + jnp.exp(-logit))
    p_ref[...] = p

    lab = lab_ref[...]
    ploss = jnp.sum(-(lab * jnp.log(p + 1e-6) + (1.0 - lab) * jnp.log(1.0 - p + 1e-6)),
                    axis=(0, 1), keepdims=True)

    @pl.when(i == 0)
    def _init():
        loss_ref[...] = jnp.zeros((1, 1), jnp.float32)

    loss_ref[...] += ploss

    @pl.when(i == _NBLK - 1)
    def _final():
        loss_ref[...] = loss_ref[...] * (1.0 / _B)


def _tc_main(uw, iw, age, gen, occ, kind, label,
             age_table, gender_table, occ_table, kind_table,
             W1, b1, W2, b2, W3, b3, W4, b4):
    bspec = lambda shp: pl.BlockSpec(shp, lambda i: (i, 0))
    fspec = lambda shp: pl.BlockSpec(shp, lambda i: (0, 0))
    grid_spec = pl.GridSpec(
        grid=(_NBLK,),
        in_specs=[
            bspec((_BB, _ED)), bspec((_BB, _ED)),
            bspec((_BB, 1)), bspec((_BB, 1)), bspec((_BB, 1)), bspec((_BB, 10)),
            bspec((_BB, 1)),
            fspec((8, _ED)), fspec((3, _ED)), fspec((25, _ED)), fspec((20, _ED)),
            fspec((960, 128)), fspec((1, 128)),
            fspec((128, 64)), fspec((1, 64)),
            fspec((64, 32)), fspec((1, 32)),
            fspec((32, 1)), fspec((1, 1)),
        ],
        out_specs=[
            fspec((1, 1)),
            bspec((_BB, 1)),
        ],
    )
    loss, p = pl.pallas_call(
        _tc_body,
        grid_spec=grid_spec,
        out_shape=[
            jax.ShapeDtypeStruct((1, 1), jnp.float32),
            jax.ShapeDtypeStruct((_B, 1), jnp.float32),
        ],
    )(uw, iw, age, gen, occ, kind, label,
      age_table, gender_table, occ_table, kind_table,
      W1, b1, W2, b2, W3, b3, W4, b4)
    return loss, p


def kernel(userid, itemid, user_age, gender, user_occupation, item_kind, label,
           user_table, item_table, age_table, gender_table, occ_table, kind_table,
           W1, b1, W2, b2, W3, b3, W4, b4):
    uidx = userid.reshape(_B).astype(jnp.int32)
    iidx = itemid.reshape(_B).astype(jnp.int32)
    sc_gather = _make_sc_gather()
    uw_flat, iw_flat = sc_gather(uidx, iidx,
                                 user_table.T.reshape(-1), item_table.T.reshape(-1))
    nc, ns = _sc_dims()
    nw = nc * ns
    bpw = _B // nw
    uw = uw_flat.reshape(nw, _ED, bpw).transpose(0, 2, 1).reshape(_B, _ED)
    iw = iw_flat.reshape(nw, _ED, bpw).transpose(0, 2, 1).reshape(_B, _ED)

    loss, p = _tc_main(
        uw, iw,
        user_age.astype(jnp.int32), gender.astype(jnp.int32),
        user_occupation.astype(jnp.int32), item_kind.astype(jnp.int32),
        label,
        age_table, gender_table, occ_table, kind_table,
        W1, b1.reshape(1, 128), W2, b2.reshape(1, 64),
        W3, b3.reshape(1, 32), W4, b4.reshape(1, 1),
    )
    return loss.reshape(()), p


# speedup vs baseline: 6.0509x; 1.3541x over previous
import functools

import jax
import jax.numpy as jnp
from jax import lax
from jax.experimental import pallas as pl
from jax.experimental.pallas import tpu as pltpu
from jax.experimental.pallas import tpu_sc as plsc

_B = 4096
_ED = 65
_DIM = 64
_BB = 512
_NBLK = _B // _BB
_NROWS = 100000


def _sc_dims():
    try:
        info = plsc.get_sparse_core_info()
        return info.num_cores, info.num_subcores
    except Exception:
        return 2, 16


def _make_sc_gather():
    nc, ns = _sc_dims()
    nw = nc * ns
    bpw = _B // nw
    epw = bpw * _ED
    nchunk = epw // 16

    mesh = plsc.VectorSubcoreMesh(core_axis_name="c", subcore_axis_name="s")

    @functools.partial(
        pl.kernel,
        mesh=mesh,
        out_type=jax.ShapeDtypeStruct((_ED * _B,), jnp.float32),
        scratch_types=[
            pltpu.VMEM((bpw,), jnp.int32),
            pltpu.VMEM((epw,), jnp.int32),
            pltpu.VMEM((epw,), jnp.float32),
            pltpu.SemaphoreType.DMA,
            pltpu.SemaphoreType.DMA,
        ],
        compiler_params=pltpu.CompilerParams(needs_layout_passes=False),
    )
    def sc_gather(idx_hbm, tab_hbm, out_hbm, idx_v, eidx_v, rows_v, gsem, osem):
        wid = lax.axis_index("s") * nc + lax.axis_index("c")
        base = wid * bpw
        pltpu.sync_copy(idx_hbm.at[pl.ds(base, bpw)], idx_v)

        iota = lax.iota(jnp.int32, 16)

        def build(i, _):
            q = lax.add(lax.broadcast(lax.mul(i, 16), (16,)), iota)
            c = lax.div(q, lax.broadcast(bpw, (16,)))
            j = lax.sub(q, lax.mul(c, lax.broadcast(bpw, (16,))))
            rv = plsc.load_gather(idx_v, [j])
            eidx_v[pl.ds(i * 16, 16)] = lax.add(
                lax.mul(c, lax.broadcast(_NROWS, (16,))), rv)
            return 0

        lax.fori_loop(0, nchunk, build, 0)
        gathers = [
            pltpu.async_copy(
                tab_hbm.at[eidx_v.at[pl.ds(c * bpw, bpw)]],
                rows_v.at[pl.ds(c * bpw, bpw)], gsem)
            for c in range(_ED)
        ]
        for g in gathers:
            g.wait()
        outs = [
            pltpu.async_copy(
                rows_v.at[pl.ds(c * bpw, bpw)],
                out_hbm.at[pl.ds(c * _B + base, bpw)], osem)
            for c in range(_ED)
        ]
        for o in outs:
            o.wait()

    return sc_gather


def _tc_body(uwT_ref, iwT_ref, ageT_ref, genT_ref, occT_ref, kidT_ref, labT_ref,
             atT_ref, gtT_ref, otT_ref, ktT_ref,
             w1T_ref, b1_ref, w2T_ref, b2_ref, w3T_ref, b3_ref, w4T_ref, b4_ref,
             loss_ref, pT_ref):
    i = pl.program_id(0)
    f32 = jnp.float32

    uT = uwT_ref[...]
    iT = iwT_ref[...]

    aohT = (ageT_ref[...] == lax.broadcasted_iota(jnp.int32, (8, _BB), 0)).astype(f32)
    gohT = (genT_ref[...] == lax.broadcasted_iota(jnp.int32, (3, _BB), 0)).astype(f32)
    oohT = (occT_ref[...] == lax.broadcasted_iota(jnp.int32, (25, _BB), 0)).astype(f32)
    awT = jnp.dot(atT_ref[...], aohT, preferred_element_type=f32)
    gwT = jnp.dot(gtT_ref[...], gohT, preferred_element_type=f32)
    owT = jnp.dot(otT_ref[...], oohT, preferred_element_type=f32)

    kidT = kidT_ref[...]
    kiota = lax.broadcasted_iota(jnp.int32, (20, _BB), 0)
    ktT = ktT_ref[...]
    kwTs = []
    for j in range(10):
        kj = kidT[j:j + 1, :]
        kohT = ((kj == kiota) & (kj != 0)).astype(f32)
        kwTs.append(jnp.dot(ktT, kohT, preferred_element_type=f32))

    fieldsT = [uT, iT, awT, owT] + kwTs + [gwT]
    oneT = fieldsT[0][0:1, :]
    for fld in fieldsT[1:]:
        oneT = oneT + fld[0:1, :]

    esT = [fld[1:, :] for fld in fieldsT]
    sT = esT[0]
    for e in esT[1:]:
        sT = sT + e
    sumsqT = jnp.sum(esT[0] * esT[0], axis=0, keepdims=True)
    for e in esT[1:]:
        sumsqT = sumsqT + jnp.sum(e * e, axis=0, keepdims=True)
    twoT = 0.5 * (jnp.sum(sT * sT, axis=0, keepdims=True) - sumsqT)

    h0T = jnp.concatenate(esT, axis=0)
    hT = jnp.maximum(jnp.dot(w1T_ref[...], h0T, preferred_element_type=f32) + b1_ref[...], 0.0)
    hT = jnp.maximum(jnp.dot(w2T_ref[...], hT, preferred_element_type=f32) + b2_ref[...], 0.0)
    hT = jnp.maximum(jnp.dot(w3T_ref[...], hT, preferred_element_type=f32) + b3_ref[...], 0.0)
    mT = jnp.dot(w4T_ref[...], hT, preferred_element_type=f32) + b4_ref[...]

    logitT = oneT + twoT + mT
    pT = 1.0 / (1.0 + jnp.exp(-logitT))
    pT_ref[...] = pT

    labT = labT_ref[...]
    ploss = jnp.sum(-(labT * jnp.log(pT + 1e-6)
                      + (1.0 - labT) * jnp.log(1.0 - pT + 1e-6)),
                    axis=(0, 1), keepdims=True)

    @pl.when(i == 0)
    def _init():
        loss_ref[...] = jnp.zeros((1, 1), jnp.float32)

    loss_ref[...] += ploss

    @pl.when(i == _NBLK - 1)
    def _final():
        loss_ref[...] = loss_ref[...] * (1.0 / _B)


def _tc_main(uwT, iwT, ageT, genT, occT, kidT, labT,
             atT, gtT, otT, ktT,
             W1T, b1, W2T, b2, W3T, b3, W4T, b4):
    bspec = lambda shp: pl.BlockSpec(shp, lambda i: (0, i))
    fspec = lambda shp: pl.BlockSpec(shp, lambda i: (0, 0))
    grid_spec = pl.GridSpec(
        grid=(_NBLK,),
        in_specs=[
            bspec((_ED, _BB)), bspec((_ED, _BB)),
            bspec((1, _BB)), bspec((1, _BB)), bspec((1, _BB)), bspec((10, _BB)),
            bspec((1, _BB)),
            fspec((_ED, 8)), fspec((_ED, 3)), fspec((_ED, 25)), fspec((_ED, 20)),
            fspec((128, 960)), fspec((128, 1)),
            fspec((64, 128)), fspec((64, 1)),
            fspec((32, 64)), fspec((32, 1)),
            fspec((1, 32)), fspec((1, 1)),
        ],
        out_specs=[
            fspec((1, 1)),
            bspec((1, _BB)),
        ],
    )
    loss, pT = pl.pallas_call(
        _tc_body,
        grid_spec=grid_spec,
        out_shape=[
            jax.ShapeDtypeStruct((1, 1), jnp.float32),
            jax.ShapeDtypeStruct((1, _B), jnp.float32),
        ],
    )(uwT, iwT, ageT, genT, occT, kidT, labT,
      atT, gtT, otT, ktT,
      W1T, b1, W2T, b2, W3T, b3, W4T, b4)
    return loss, pT


def kernel(userid, itemid, user_age, gender, user_occupation, item_kind, label,
           user_table, item_table, age_table, gender_table, occ_table, kind_table,
           W1, b1, W2, b2, W3, b3, W4, b4):
    uidx = userid.reshape(_B).astype(jnp.int32)
    iidx = itemid.reshape(_B).astype(jnp.int32)
    sc_gather = _make_sc_gather()
    uwT = sc_gather(uidx, user_table.T.reshape(-1)).reshape(_ED, _B)
    iwT = sc_gather(iidx, item_table.T.reshape(-1)).reshape(_ED, _B)

    loss, pT = _tc_main(
        uwT, iwT,
        user_age.astype(jnp.int32).reshape(1, _B),
        gender.astype(jnp.int32).reshape(1, _B),
        user_occupation.astype(jnp.int32).reshape(1, _B),
        item_kind.astype(jnp.int32).T,
        label.reshape(1, _B),
        age_table.T, gender_table.T, occ_table.T, kind_table.T,
        W1.T, b1.reshape(128, 1), W2.T, b2.reshape(64, 1),
        W3.T, b3.reshape(32, 1), W4.T, b4.reshape(1, 1),
    )
    return loss.reshape(()), pT.reshape(_B, 1)


# speedup vs baseline: 6.2118x; 1.0266x over previous
import functools

import jax
import jax.numpy as jnp
from jax import lax
from jax.experimental import pallas as pl
from jax.experimental.pallas import tpu as pltpu
from jax.experimental.pallas import tpu_sc as plsc

_B = 4096
_ED = 65
_DIM = 64
_BB = 1024
_NBLK = _B // _BB
_NROWS = 100000


def _sc_dims():
    try:
        info = plsc.get_sparse_core_info()
        return info.num_cores, info.num_subcores
    except Exception:
        return 2, 16


def _make_sc_gather():
    nc, ns = _sc_dims()
    nw = nc * ns
    bpw = _B // nw
    epw = bpw * _ED
    nchunk = epw // 16

    mesh = plsc.VectorSubcoreMesh(core_axis_name="c", subcore_axis_name="s")

    @functools.partial(
        pl.kernel,
        mesh=mesh,
        out_type=jax.ShapeDtypeStruct((_ED * _B,), jnp.float32),
        scratch_types=[
            pltpu.VMEM((bpw,), jnp.int32),
            pltpu.VMEM((epw,), jnp.int32),
            pltpu.VMEM((epw,), jnp.float32),
            pltpu.SemaphoreType.DMA,
            pltpu.SemaphoreType.DMA,
        ],
        compiler_params=pltpu.CompilerParams(needs_layout_passes=False),
    )
    def sc_gather(idx_hbm, tab_hbm, out_hbm, idx_v, eidx_v, rows_v, gsem, osem):
        wid = lax.axis_index("s") * nc + lax.axis_index("c")
        base = wid * bpw
        pltpu.sync_copy(idx_hbm.at[pl.ds(base, bpw)], idx_v)

        iota = lax.iota(jnp.int32, 16)

        def build(i, _):
            q = lax.add(lax.broadcast(lax.mul(i, 16), (16,)), iota)
            c = lax.div(q, lax.broadcast(bpw, (16,)))
            j = lax.sub(q, lax.mul(c, lax.broadcast(bpw, (16,))))
            rv = plsc.load_gather(idx_v, [j])
            eidx_v[pl.ds(i * 16, 16)] = lax.add(
                lax.mul(c, lax.broadcast(_NROWS, (16,))), rv)
            return 0

        lax.fori_loop(0, nchunk, build, 0)
        gathers = [
            pltpu.async_copy(
                tab_hbm.at[eidx_v.at[pl.ds(c * bpw, bpw)]],
                rows_v.at[pl.ds(c * bpw, bpw)], gsem)
            for c in range(_ED)
        ]
        for g in gathers:
            g.wait()
        outs = [
            pltpu.async_copy(
                rows_v.at[pl.ds(c * bpw, bpw)],
                out_hbm.at[pl.ds(c * _B + base, bpw)], osem)
            for c in range(_ED)
        ]
        for o in outs:
            o.wait()

    return sc_gather


def _tc_body(uwT_ref, iwT_ref, ageT_ref, genT_ref, occT_ref, kidT_ref, labT_ref,
             atT_ref, gtT_ref, otT_ref, ktT_ref,
             w1T_ref, b1_ref, w2T_ref, b2_ref, w3T_ref, b3_ref, w4T_ref, b4_ref,
             loss_ref, pT_ref):
    i = pl.program_id(0)
    f32 = jnp.float32

    uT = uwT_ref[...]
    iT = iwT_ref[...]

    aohT = (ageT_ref[...] == lax.broadcasted_iota(jnp.int32, (8, _BB), 0)).astype(f32)
    gohT = (genT_ref[...] == lax.broadcasted_iota(jnp.int32, (3, _BB), 0)).astype(f32)
    oohT = (occT_ref[...] == lax.broadcasted_iota(jnp.int32, (25, _BB), 0)).astype(f32)
    awT = jnp.dot(atT_ref[...], aohT, preferred_element_type=f32)
    gwT = jnp.dot(gtT_ref[...], gohT, preferred_element_type=f32)
    owT = jnp.dot(otT_ref[...], oohT, preferred_element_type=f32)

    kidT = kidT_ref[...]
    kiota = lax.broadcasted_iota(jnp.int32, (20, _BB), 0)
    ktT = ktT_ref[...]
    kwTs = []
    for j in range(10):
        kj = kidT[j:j + 1, :]
        kohT = ((kj == kiota) & (kj != 0)).astype(f32)
        kwTs.append(jnp.dot(ktT, kohT, preferred_element_type=f32))

    fieldsT = [uT, iT, awT, owT] + kwTs + [gwT]
    oneT = fieldsT[0][0:1, :]
    for fld in fieldsT[1:]:
        oneT = oneT + fld[0:1, :]

    esT = [fld[1:, :] for fld in fieldsT]
    sT = esT[0]
    for e in esT[1:]:
        sT = sT + e
    sumsqT = jnp.sum(esT[0] * esT[0], axis=0, keepdims=True)
    for e in esT[1:]:
        sumsqT = sumsqT + jnp.sum(e * e, axis=0, keepdims=True)
    twoT = 0.5 * (jnp.sum(sT * sT, axis=0, keepdims=True) - sumsqT)

    h0T = jnp.concatenate(esT, axis=0)
    hT = jnp.maximum(jnp.dot(w1T_ref[...], h0T, preferred_element_type=f32) + b1_ref[...], 0.0)
    hT = jnp.maximum(jnp.dot(w2T_ref[...], hT, preferred_element_type=f32) + b2_ref[...], 0.0)
    hT = jnp.maximum(jnp.dot(w3T_ref[...], hT, preferred_element_type=f32) + b3_ref[...], 0.0)
    mT = jnp.dot(w4T_ref[...], hT, preferred_element_type=f32) + b4_ref[...]

    logitT = oneT + twoT + mT
    pT = 1.0 / (1.0 + jnp.exp(-logitT))
    pT_ref[...] = pT

    labT = labT_ref[...]
    ploss = jnp.sum(-(labT * jnp.log(pT + 1e-6)
                      + (1.0 - labT) * jnp.log(1.0 - pT + 1e-6)),
                    axis=(0, 1), keepdims=True)

    @pl.when(i == 0)
    def _init():
        loss_ref[...] = jnp.zeros((1, 1), jnp.float32)

    loss_ref[...] += ploss

    @pl.when(i == _NBLK - 1)
    def _final():
        loss_ref[...] = loss_ref[...] * (1.0 / _B)


def _tc_main(uwT, iwT, ageT, genT, occT, kidT, labT,
             atT, gtT, otT, ktT,
             W1T, b1, W2T, b2, W3T, b3, W4T, b4):
    bspec = lambda shp: pl.BlockSpec(shp, lambda i: (0, i))
    fspec = lambda shp: pl.BlockSpec(shp, lambda i: (0, 0))
    grid_spec = pl.GridSpec(
        grid=(_NBLK,),
        in_specs=[
            bspec((_ED, _BB)), bspec((_ED, _BB)),
            bspec((1, _BB)), bspec((1, _BB)), bspec((1, _BB)), bspec((10, _BB)),
            bspec((1, _BB)),
            fspec((_ED, 8)), fspec((_ED, 3)), fspec((_ED, 25)), fspec((_ED, 20)),
            fspec((128, 960)), fspec((128, 1)),
            fspec((64, 128)), fspec((64, 1)),
            fspec((32, 64)), fspec((32, 1)),
            fspec((1, 32)), fspec((1, 1)),
        ],
        out_specs=[
            fspec((1, 1)),
            bspec((1, _BB)),
        ],
    )
    loss, pT = pl.pallas_call(
        _tc_body,
        grid_spec=grid_spec,
        out_shape=[
            jax.ShapeDtypeStruct((1, 1), jnp.float32),
            jax.ShapeDtypeStruct((1, _B), jnp.float32),
        ],
    )(uwT, iwT, ageT, genT, occT, kidT, labT,
      atT, gtT, otT, ktT,
      W1T, b1, W2T, b2, W3T, b3, W4T, b4)
    return loss, pT


def kernel(userid, itemid, user_age, gender, user_occupation, item_kind, label,
           user_table, item_table, age_table, gender_table, occ_table, kind_table,
           W1, b1, W2, b2, W3, b3, W4, b4):
    uidx = userid.reshape(_B).astype(jnp.int32)
    iidx = itemid.reshape(_B).astype(jnp.int32)
    sc_gather = _make_sc_gather()
    uwT = sc_gather(uidx, user_table.T.reshape(-1)).reshape(_ED, _B)
    iwT = sc_gather(iidx, item_table.T.reshape(-1)).reshape(_ED, _B)

    loss, pT = _tc_main(
        uwT, iwT,
        user_age.astype(jnp.int32).reshape(1, _B),
        gender.astype(jnp.int32).reshape(1, _B),
        user_occupation.astype(jnp.int32).reshape(1, _B),
        item_kind.astype(jnp.int32).T,
        label.reshape(1, _B),
        age_table.T, gender_table.T, occ_table.T, kind_table.T,
        W1.T, b1.reshape(128, 1), W2.T, b2.reshape(64, 1),
        W3.T, b3.reshape(32, 1), W4.T, b4.reshape(1, 1),
    )
    return loss.reshape(()), pT.reshape(_B, 1)
